# baseline (speedup 1.0000x reference)
import jax
import jax.numpy as jnp
from jax import lax
from jax.experimental import pallas as pl
from jax.experimental.pallas import tpu as pltpu
from jax.experimental.pallas import tpu_sc as plsc

B = 16384
D = 4096
K = 128
NC, NS = 2, 16
NW = NC * NS
ROWS_PER_W = B // NW
IDX_PER_W = ROWS_PER_W * K
GN = IDX_PER_W // 2
TCR = 512


def _copy_body(x_ref, o_ref):
    o_ref[...] = x_ref[...]


_tc_copy = pl.pallas_call(
    _copy_body,
    grid=(B // TCR,),
    in_specs=[pl.BlockSpec((TCR, D), lambda i: (i, 0))],
    out_specs=pl.BlockSpec((TCR, D), lambda i: (i, 0)),
    out_shape=jax.ShapeDtypeStruct((B, D), jnp.float32),
)


def _sc_body(out_hbm, idx_hbm, idx0_v, idx1_v, vals_v, sem_idx, sem_sc):
    wid = lax.axis_index("s") * NC + lax.axis_index("c")
    base = wid * IDX_PER_W

    in0 = pltpu.make_async_copy(idx_hbm.at[pl.ds(base, GN)], idx0_v, sem_idx)
    in1 = pltpu.make_async_copy(
        idx_hbm.at[pl.ds(base + GN, GN)], idx1_v, sem_idx)
    in0.start()
    in1.start()

    half = jnp.full((16,), 0.5, dtype=jnp.float32)

    @pl.loop(0, GN // 16)
    def _fill(j):
        vals_v[pl.ds(j * 16, 16)] = half

    in0.wait()
    in1.wait()

    sc0 = pltpu.make_async_copy(vals_v, out_hbm.at[idx0_v], sem_sc)
    sc1 = pltpu.make_async_copy(vals_v, out_hbm.at[idx1_v], sem_sc)
    sc0.start()
    sc1.start()
    sc0.wait()
    sc1.wait()


_mesh = plsc.VectorSubcoreMesh(
    core_axis_name="c", subcore_axis_name="s", num_cores=NC, num_subcores=NS)

_sc_scatter = pl.kernel(
    _sc_body,
    mesh=_mesh,
    compiler_params=pltpu.CompilerParams(needs_layout_passes=False),
    scratch_types=[
        pltpu.VMEM((GN,), jnp.int32),
        pltpu.VMEM((GN,), jnp.int32),
        pltpu.VMEM((GN,), jnp.float32),
        pltpu.SemaphoreType.DMA,
        pltpu.SemaphoreType.DMA,
    ],
)


def kernel(x, index):
    rows = jnp.arange(B, dtype=jnp.int32)[:, None]
    flat_idx = (rows * D + index.astype(jnp.int32)).reshape(B * K)
    copied = _tc_copy(x)
    buf = jax.new_ref(copied.reshape(B * D))
    _sc_scatter(buf, flat_idx)
    return buf[...].reshape(B, D)


# speedup vs baseline: 2.5933x; 2.5933x over previous
import jax
import jax.numpy as jnp
from jax import lax
from jax.experimental import pallas as pl
from jax.experimental.pallas import tpu as pltpu
from jax.experimental.pallas import tpu_sc as plsc

B = 16384
D = 4096
K = 128
NC, NS = 2, 16
NW = NC * NS
ROWS_PER_W = B // NW

R = 8
R2_ROWS = 400
R2_CHUNKS = R2_ROWS // R

BR = 4
BULK_CHUNKS = (ROWS_PER_W - R2_ROWS) // BR
BN = BR * K

VPR = K // 16


def _body(x_hbm, idx_hbm, out_hbm, *refs):
    (spmem, data0, data1, idxb0, idxb1) = refs[:5]
    bidx = refs[5:5 + BULK_CHUNKS]
    vals_v = refs[5 + BULK_CHUNKS]
    (sd0, sd1, si0, si1, so0, so1,
     sbh0, sbh1, sbs0, sbs1, sbi, ssc) = refs[6 + BULK_CHUNKS:]

    c = lax.axis_index("c")
    s = lax.axis_index("s")
    wid = s * NC + c
    row0 = wid * ROWS_PER_W
    brow0 = row0 + R2_ROWS

    data = (data0, data1)
    idxb = (idxb0, idxb1)
    sd = (sd0, sd1)
    si = (si0, si1)
    so = (so0, so1)
    sbh = (sbh0, sbh1)
    sbs = (sbs0, sbs1)

    half = jnp.full((16,), 0.5, dtype=jnp.float32)

    def in_copies(g, b):
        r = row0 + g * R
        return (
            pltpu.make_async_copy(
                x_hbm.at[pl.ds(r * D, R * D)], data[b], sd[b]),
            pltpu.make_async_copy(
                idx_hbm.at[pl.ds(r * K, R * K)], idxb[b], si[b]),
        )

    def out_copy(g, b):
        r = row0 + g * R
        return pltpu.make_async_copy(
            data[b], out_hbm.at[pl.ds(r * D, R * D)], so[b])

    def start_in(g, b):
        a, d = in_copies(g, b)
        a.start()
        d.start()

    def wait_in(g, b):
        a, d = in_copies(g, b)
        a.wait()
        d.wait()

    def scatter(g, b):
        base = (row0 + g * R) * D
        for r in range(R):
            for j in range(VPR):
                flat = idxb[b][pl.ds(r * K + j * 16, 16)]
                plsc.store_scatter(data[b], [flat - base], half)

    def bh2s(k):
        r = brow0 + k * BR
        return pltpu.make_async_copy(
            x_hbm.at[pl.ds(r * D, BR * D)], spmem.at[s, k % 2], sbh[k % 2])

    def bs2h(k):
        r = brow0 + k * BR
        return pltpu.make_async_copy(
            spmem.at[s, k % 2], out_hbm.at[pl.ds(r * D, BR * D)], sbs[k % 2])

    def bidx_cp(k):
        r = brow0 + k * BR
        return pltpu.make_async_copy(
            idx_hbm.at[pl.ds(r * K, BN)], bidx[k], sbi)

    def fire(k):
        return pltpu.make_async_copy(vals_v, out_hbm.at[bidx[k]], ssc)

    start_in(0, 0)
    start_in(1, 1)

    for k in range(BULK_CHUNKS):
        bidx_cp(k).start()
    bh2s(0).start()
    bh2s(1).start()

    @pl.loop(0, BN // 16)
    def _fill(j):
        vals_v[pl.ds(j * 16, 16)] = half

    for k in range(BULK_CHUNKS):
        bidx_cp(k).wait()

    for k in range(BULK_CHUNKS):
        if k >= 2:
            bs2h(k - 2).wait()
            fire(k - 2).start()
            bh2s(k).start()
        bh2s(k).wait()
        bs2h(k).start()
    bs2h(BULK_CHUNKS - 2).wait()
    fire(BULK_CHUNKS - 2).start()
    bs2h(BULK_CHUNKS - 1).wait()
    fire(BULK_CHUNKS - 1).start()

    wait_in(0, 0)
    scatter(0, 0)
    out_copy(0, 0).start()
    out_copy(0, 0).wait()
    start_in(2, 0)
    wait_in(1, 1)
    scatter(1, 1)
    out_copy(1, 1).start()

    def loop_body(i, carry):
        g0 = i * 2
        out_copy(g0 - 1, 1).wait()
        start_in(g0 + 1, 1)
        wait_in(g0, 0)
        scatter(g0, 0)
        out_copy(g0, 0).start()
        out_copy(g0, 0).wait()
        start_in(g0 + 2, 0)
        wait_in(g0 + 1, 1)
        scatter(g0 + 1, 1)
        out_copy(g0 + 1, 1).start()
        return carry

    lax.fori_loop(1, R2_CHUNKS // 2 - 1, loop_body, None)

    g = R2_CHUNKS - 2
    out_copy(g - 1, 1).wait()
    start_in(g + 1, 1)
    wait_in(g, 0)
    scatter(g, 0)
    out_copy(g, 0).start()
    out_copy(g, 0).wait()
    wait_in(g + 1, 1)
    scatter(g + 1, 1)
    out_copy(g + 1, 1).start()
    out_copy(g + 1, 1).wait()

    for k in range(BULK_CHUNKS):
        fire(k).wait()


_mesh = plsc.VectorSubcoreMesh(
    core_axis_name="c", subcore_axis_name="s", num_cores=NC, num_subcores=NS)

_scatter_call = pl.kernel(
    _body,
    out_type=jax.ShapeDtypeStruct((B * D,), jnp.float32),
    mesh=_mesh,
    compiler_params=pltpu.CompilerParams(needs_layout_passes=False),
    scratch_types=(
        [pltpu.VMEM_SHARED((NS, 2, BR * D), jnp.float32)]
        + [pltpu.VMEM((R * D,), jnp.float32)] * 2
        + [pltpu.VMEM((R * K,), jnp.int32)] * 2
        + [pltpu.VMEM((BN,), jnp.int32)] * BULK_CHUNKS
        + [pltpu.VMEM((BN,), jnp.float32)]
        + [pltpu.SemaphoreType.DMA] * 12
    ),
)


def kernel(x, index):
    rows = jnp.arange(B, dtype=jnp.int32)[:, None]
    flat_idx = (rows * D + index.astype(jnp.int32)).reshape(B * K)
    flat = _scatter_call(x.reshape(B * D), flat_idx)
    return flat.reshape(B, D)


# speedup vs baseline: 4.5385x; 1.7501x over previous
import jax
import jax.numpy as jnp
from jax import lax
from jax.experimental import pallas as pl
from jax.experimental.pallas import tpu as pltpu
from jax.experimental.pallas import tpu_sc as plsc

B = 16384
D = 4096
K = 128
NC, NS = 2, 16
NW = NC * NS
ROWS_PER_W = B // NW
R = 8
CHUNKS = ROWS_PER_W // R
VPR = K // 16


def _body(x_hbm, idx_hbm, out_hbm,
          data0, data1, data2, idxb0, idxb1, idxb2,
          sd0, sd1, sd2, si0, si1, si2, so0, so1, so2):
    wid = lax.axis_index("s") * NC + lax.axis_index("c")
    row0 = wid * ROWS_PER_W

    data = (data0, data1, data2)
    idxb = (idxb0, idxb1, idxb2)
    sd = (sd0, sd1, sd2)
    si = (si0, si1, si2)
    so = (so0, so1, so2)

    half = jnp.full((16,), 0.5, dtype=jnp.float32)

    def in_copies(g, b):
        r = row0 + g * R
        return (
            pltpu.make_async_copy(
                x_hbm.at[pl.ds(r * D, R * D)], data[b], sd[b]),
            pltpu.make_async_copy(
                idx_hbm.at[pl.ds(r * K, R * K)], idxb[b], si[b]),
        )

    def out_copy(g, b):
        r = row0 + g * R
        return pltpu.make_async_copy(
            data[b], out_hbm.at[pl.ds(r * D, R * D)], so[b])

    def start_in(g, b):
        a, c = in_copies(g, b)
        a.start()
        c.start()

    def wait_in(g, b):
        a, c = in_copies(g, b)
        a.wait()
        c.wait()

    def scatter(b):
        for r in range(R):
            for j in range(VPR):
                cols = idxb[b][pl.ds(r * K + j * 16, 16)]
                plsc.store_scatter(data[b], [cols + r * D], half)

    start_in(0, 0)
    start_in(1, 1)
    start_in(2, 2)
    wait_in(0, 0)
    scatter(0)
    out_copy(0, 0).start()
    wait_in(1, 1)
    scatter(1)
    out_copy(1, 1).start()
    out_copy(0, 0).wait()
    start_in(3, 0)
    wait_in(2, 2)
    scatter(2)
    out_copy(2, 2).start()

    def loop_body(i, carry):
        g0 = i * 3
        for k in range(3):
            g = g0 + k
            b = k
            nb = (k + 1) % 3
            out_copy(g - 2, nb).wait()
            start_in(g + 1, nb)
            wait_in(g, b)
            scatter(b)
            out_copy(g, b).start()
        return carry

    lax.fori_loop(1, (CHUNKS - 1) // 3, loop_body, None)

    g = CHUNKS - 1
    out_copy(g - 2, 1).wait()
    wait_in(g, 0)
    scatter(0)
    out_copy(g, 0).start()
    out_copy(g - 1, 2).wait()
    out_copy(g, 0).wait()


_mesh = plsc.VectorSubcoreMesh(
    core_axis_name="c", subcore_axis_name="s", num_cores=NC, num_subcores=NS)

_scatter_call = pl.kernel(
    _body,
    out_type=jax.ShapeDtypeStruct((B * D,), jnp.float32),
    mesh=_mesh,
    compiler_params=pltpu.CompilerParams(needs_layout_passes=False),
    scratch_types=(
        [pltpu.VMEM((R * D,), jnp.float32)] * 3
        + [pltpu.VMEM((R * K,), jnp.int32)] * 3
        + [pltpu.SemaphoreType.DMA] * 9
    ),
)


def kernel(x, index):
    flat = _scatter_call(
        x.reshape(B * D), index.astype(jnp.int32).reshape(B * K))
    return flat.reshape(B, D)
